# baseline (speedup 1.0000x reference)
import functools

import jax
import jax.numpy as jnp
from jax import lax
from jax.experimental import pallas as pl
from jax.experimental.pallas import tpu as pltpu
from jax.experimental.pallas import tpu_sc as plsc

N = 10000
E = 160000
D_IN = 256
D_HID = 256
D_OUT = 128

NC = 2
NS = 16
CHUNK = 96
E0P = 16 * 105 * CHUNK
E1P = 32 * 53 * CHUNK
NP = 10240
ROWS_PT = NP // NS

RBLK = 1000
NB = N // RBLK


@functools.lru_cache(maxsize=None)
def _make_sc_agg(dh, chunk, feature_split):
  mesh = plsc.VectorSubcoreMesh(core_axis_name="c", subcore_axis_name="s",
                                num_cores=NC, num_subcores=NS)
  out_type = jax.ShapeDtypeStruct((NC * NP, dh), jnp.float32)
  scratch = [
      pltpu.VMEM((chunk,), jnp.int32),
      pltpu.VMEM((chunk,), jnp.int32),
      pltpu.VMEM((chunk, dh), jnp.float32),
      pltpu.VMEM((chunk,), jnp.int32),
      pltpu.VMEM((chunk,), jnp.int32),
      pltpu.VMEM((chunk, dh), jnp.float32),
      pltpu.VMEM_SHARED((NP, dh), jnp.float32),
      pltpu.SemaphoreType.DMA,
      pltpu.SemaphoreType.DMA,
  ]
  ept = E0P // NS if feature_split else E1P // (NC * NS)
  nchunk = ept // chunk
  assert ept % chunk == 0

  pairs = (nchunk - 1) // 2
  assert nchunk == 2 * pairs + 1

  def body(table, src, dst, zacc, out,
           sb0, db0, rb0, sb1, db1, rb1, acc, sem0, sem1):
    c = lax.axis_index("c")
    s = lax.axis_index("s")
    rbase = s * ROWS_PT

    pltpu.sync_copy(zacc.at[pl.ds(rbase, ROWS_PT)], acc.at[pl.ds(rbase, ROWS_PT)])
    plsc.subcore_barrier()

    if feature_split:
      ebase = s * ept
      roff = c * N
    else:
      ebase = (c * NS + s) * ept

    def start(i, sb, db, rb, sem):
      off = ebase + i * chunk
      pltpu.sync_copy(src.at[pl.ds(off, chunk)], sb)
      pltpu.sync_copy(dst.at[pl.ds(off, chunk)], db)
      if feature_split:
        for j in range(chunk // 16):
          sl = pl.ds(j * 16, 16)
          sb[sl] = sb[sl] + roff
      pltpu.async_copy(table.at[sb], rb, sem)

    def drain(sb, db, rb, sem):
      pltpu.make_async_copy(table.at[sb], rb, sem).wait()
      pltpu.sync_copy(rb, acc.at[db], add=True)

    start(0, sb0, db0, rb0, sem0)

    def pair_body(k, carry):
      start(2 * k + 1, sb1, db1, rb1, sem1)
      drain(sb0, db0, rb0, sem0)
      start(2 * k + 2, sb0, db0, rb0, sem0)
      drain(sb1, db1, rb1, sem1)
      return carry

    lax.fori_loop(0, pairs, pair_body, 0)
    drain(sb0, db0, rb0, sem0)
    plsc.subcore_barrier()

    pltpu.sync_copy(acc.at[pl.ds(rbase, ROWS_PT)],
                    out.at[pl.ds(c * NP + rbase, ROWS_PT)])

  return functools.partial(
      pl.kernel, mesh=mesh, out_type=out_type, scratch_types=scratch
  )(body)


CCHUNK = 96


@functools.lru_cache(maxsize=None)
def _make_sc_cnt():
  mesh = plsc.VectorSubcoreMesh(core_axis_name="c", subcore_axis_name="s",
                                num_cores=NC, num_subcores=NS)
  out_type = jax.ShapeDtypeStruct((NC * NP, 128), jnp.float32)
  scratch = [
      pltpu.VMEM((CCHUNK,), jnp.int32),
      pltpu.VMEM((CCHUNK,), jnp.int32),
      pltpu.VMEM((CCHUNK, 128), jnp.float32),
      pltpu.VMEM_SHARED((NP, 128), jnp.float32),
      pltpu.SemaphoreType.DMA,
      pltpu.SemaphoreType.DMA,
  ]
  ept = E1P // (NC * NS)
  nchunk = ept // CCHUNK

  pairs = (nchunk - 1) // 2
  assert nchunk == 2 * pairs + 1

  def body(dst, zacc, ones_h, out, db0, db1, ones_v, acc, sem0, sem1):
    c = lax.axis_index("c")
    s = lax.axis_index("s")
    rbase = s * ROWS_PT
    pltpu.sync_copy(zacc.at[pl.ds(rbase, ROWS_PT)], acc.at[pl.ds(rbase, ROWS_PT)])
    pltpu.sync_copy(ones_h, ones_v)
    plsc.subcore_barrier()
    ebase = (c * NS + s) * ept

    def start(i, db, sem):
      pltpu.sync_copy(dst.at[pl.ds(ebase + i * CCHUNK, CCHUNK)], db)
      pltpu.async_copy(ones_v, acc.at[db], sem, add=True)

    def drain(db, sem):
      pltpu.make_async_copy(ones_v, acc.at[db], sem).wait()

    start(0, db0, sem0)

    def pair_body(k, carry):
      start(2 * k + 1, db1, sem1)
      drain(db0, sem0)
      start(2 * k + 2, db0, sem0)
      drain(db1, sem1)
      return carry

    lax.fori_loop(0, pairs, pair_body, 0)
    drain(db0, sem0)
    plsc.subcore_barrier()
    pltpu.sync_copy(acc.at[pl.ds(rbase, ROWS_PT)],
                    out.at[pl.ds(c * NP + rbase, ROWS_PT)])

  return functools.partial(
      pl.kernel, mesh=mesh, out_type=out_type, scratch_types=scratch
  )(body)


def _tca_body(x_ref, wl_ref, wr_ref, wres_ref, resb_ref,
              ycat_ref, xr_ref, res_ref):
  xb = x_ref[...]
  y = jnp.dot(xb, wl_ref[...], preferred_element_type=jnp.float32)
  ycat_ref[0] = y[:, : D_HID // 2]
  ycat_ref[1] = y[:, D_HID // 2:]
  xr_ref[...] = jnp.dot(xb, wr_ref[...], preferred_element_type=jnp.float32)
  res_ref[...] = (jnp.dot(xb, wres_ref[...], preferred_element_type=jnp.float32)
                  + resb_ref[...])


_tca = pl.pallas_call(
    _tca_body,
    grid=(NB,),
    in_specs=[
        pl.BlockSpec((RBLK, D_IN), lambda i: (i, 0)),
        pl.BlockSpec((D_IN, D_HID), lambda i: (0, 0)),
        pl.BlockSpec((D_IN, D_HID), lambda i: (0, 0)),
        pl.BlockSpec((D_IN, D_OUT), lambda i: (0, 0)),
        pl.BlockSpec((1, D_OUT), lambda i: (0, 0)),
    ],
    out_specs=[
        pl.BlockSpec((2, RBLK, D_HID // 2), lambda i: (0, i, 0)),
        pl.BlockSpec((RBLK, D_HID), lambda i: (i, 0)),
        pl.BlockSpec((RBLK, D_OUT), lambda i: (i, 0)),
    ],
    out_shape=[
        jax.ShapeDtypeStruct((2, N, D_HID // 2), jnp.float32),
        jax.ShapeDtypeStruct((N, D_HID), jnp.float32),
        jax.ShapeDtypeStruct((N, D_OUT), jnp.float32),
    ],
)


def _tcb_body(sum_ref, cnt_ref, xr_ref, bl0_ref, g_ref, b_ref,
              wl1_ref, wr1_ref, ycat_ref, hr_ref, h_s, stat_s):
  p = pl.program_id(0)
  i = pl.program_id(1)

  @pl.when(p == 0)
  def _():
    @pl.when(i == 0)
    def _():
      stat_s[...] = jnp.zeros_like(stat_s)

    inv = 1.0 / jnp.maximum(cnt_ref[0, :, 0:1] + cnt_ref[1, :, 0:1], 1.0)
    t = (jnp.concatenate([sum_ref[0], sum_ref[1]], axis=1) * inv
         + bl0_ref[...] + xr_ref[...])
    nrm = jnp.maximum(jnp.sqrt(jnp.sum(t * t, axis=1, keepdims=True)), 1e-12)
    t = t / nrm
    h_s[pl.ds(i * RBLK, RBLK), :] = t
    stat_s[0:1, :] += jnp.sum(t, axis=0, keepdims=True)
    stat_s[1:2, :] += jnp.sum(t * t, axis=0, keepdims=True)

  @pl.when(p == 1)
  def _():
    mu = stat_s[0:1, :] / N
    var = stat_s[1:2, :] / N - mu * mu
    t = h_s[pl.ds(i * RBLK, RBLK), :]
    t = g_ref[...] * (t - mu) * lax.rsqrt(var + 1e-5) + b_ref[...]
    t = jnp.maximum(t, 0.0)
    ycat_ref[...] = jnp.dot(t, wl1_ref[...], preferred_element_type=jnp.float32)
    hr_ref[...] = jnp.dot(t, wr1_ref[...], preferred_element_type=jnp.float32)


_tcb = pl.pallas_call(
    _tcb_body,
    grid=(2, NB),
    in_specs=[
        pl.BlockSpec((2, RBLK, D_HID // 2), lambda p, i: (0, i, 0)),
        pl.BlockSpec((2, RBLK, 8), lambda p, i: (0, i, 0)),
        pl.BlockSpec((RBLK, D_HID), lambda p, i: (i, 0)),
        pl.BlockSpec((1, D_HID), lambda p, i: (0, 0)),
        pl.BlockSpec((1, D_HID), lambda p, i: (0, 0)),
        pl.BlockSpec((1, D_HID), lambda p, i: (0, 0)),
        pl.BlockSpec((D_HID, D_OUT), lambda p, i: (0, 0)),
        pl.BlockSpec((D_HID, D_OUT), lambda p, i: (0, 0)),
    ],
    out_specs=[
        pl.BlockSpec((RBLK, D_OUT), lambda p, i: (i, 0)),
        pl.BlockSpec((RBLK, D_OUT), lambda p, i: (i, 0)),
    ],
    out_shape=[
        jax.ShapeDtypeStruct((N, D_OUT), jnp.float32),
        jax.ShapeDtypeStruct((N, D_OUT), jnp.float32),
    ],
    scratch_shapes=[
        pltpu.VMEM((N, D_HID), jnp.float32),
        pltpu.VMEM((2, D_HID), jnp.float32),
    ],
)


def _tcc_body(sum_ref, cnt_ref, hr_ref, res_ref, bl1_ref, out_ref):
  inv = 1.0 / jnp.maximum(cnt_ref[0, :, 0:1] + cnt_ref[1, :, 0:1], 1.0)
  t = (sum_ref[0] + sum_ref[1]) * inv + bl1_ref[...] + hr_ref[...]
  nrm = jnp.maximum(jnp.sqrt(jnp.sum(t * t, axis=1, keepdims=True)), 1e-12)
  t = t / nrm + res_ref[...]
  nrm = jnp.maximum(jnp.sqrt(jnp.sum(t * t, axis=1, keepdims=True)), 1e-12)
  out_ref[...] = t / nrm


_tcc = pl.pallas_call(
    _tcc_body,
    grid=(NB,),
    in_specs=[
        pl.BlockSpec((2, RBLK, D_OUT), lambda i: (0, i, 0)),
        pl.BlockSpec((2, RBLK, 8), lambda i: (0, i, 0)),
        pl.BlockSpec((RBLK, D_OUT), lambda i: (i, 0)),
        pl.BlockSpec((RBLK, D_OUT), lambda i: (i, 0)),
        pl.BlockSpec((1, D_OUT), lambda i: (0, 0)),
    ],
    out_specs=pl.BlockSpec((RBLK, D_OUT), lambda i: (i, 0)),
    out_shape=jax.ShapeDtypeStruct((N, D_OUT), jnp.float32),
)


def kernel(x, edge_index, Wl0, bl0, Wr0, gamma0, beta0, Wl1, bl1, Wr1,
           resW, resb):
  src = edge_index[0]
  dst = edge_index[1]
  src0 = jnp.concatenate([src, jnp.zeros((E0P - E,), jnp.int32)])
  dst0 = jnp.concatenate([dst, jnp.full((E0P - E,), N, jnp.int32)])
  src1 = jnp.concatenate([src, jnp.zeros((E1P - E,), jnp.int32)])
  dst1 = jnp.concatenate([dst, jnp.full((E1P - E,), N, jnp.int32)])

  ycat0, xr0, res = _tca(x, Wl0.T, Wr0.T, resW.T, resb.reshape(1, -1))

  zacc = jnp.zeros((NP, 128), jnp.float32)
  ones_h = jnp.ones((CCHUNK, 128), jnp.float32)
  cntcat = _make_sc_cnt()(dst1, zacc, ones_h)
  cnt8 = cntcat.reshape(NC, NP, 128)[:, :N, :8]

  sum0 = _make_sc_agg(128, CHUNK, True)(
      ycat0.reshape(NC * N, 128), src0, dst0, zacc)
  sum0 = sum0.reshape(NC, NP, 128)[:, :N, :]

  y1, hr1 = _tcb(sum0, cnt8, xr0,
                 bl0.reshape(1, -1), gamma0.reshape(1, -1),
                 beta0.reshape(1, -1), Wl1.T, Wr1.T)

  (sum1) = _make_sc_agg(128, CCHUNK, False)(y1, src1, dst1, zacc)
  sum1 = sum1.reshape(NC, NP, 128)[:, :N, :]

  return _tcc(sum1, cnt8, hr1, res, bl1.reshape(1, -1))


# speedup vs baseline: 1.2175x; 1.2175x over previous
import functools

import jax
import jax.numpy as jnp
from jax import lax
from jax.experimental import pallas as pl
from jax.experimental.pallas import tpu as pltpu
from jax.experimental.pallas import tpu_sc as plsc

N = 10000
E = 160000
D_IN = 256
D_HID = 256
D_OUT = 128

NC = 2
NS = 16
CHUNK = 80
E1P = 32 * 63 * CHUNK
NP = 10240
ROWS_PT = NP // NS

RBLK = 1000
NB = N // RBLK


@functools.lru_cache(maxsize=None)
def _make_sc_agg(dh, chunk, feature_split):
  mesh = plsc.VectorSubcoreMesh(core_axis_name="c", subcore_axis_name="s",
                                num_cores=NC, num_subcores=NS)
  out_type = jax.ShapeDtypeStruct((NC * NP, dh), jnp.float32)
  nbuf = 3 if feature_split else 2
  scratch = []
  for _ in range(nbuf):
    scratch += [
        pltpu.VMEM((chunk,), jnp.int32),
        pltpu.VMEM((chunk,), jnp.int32),
        pltpu.VMEM((chunk, dh), jnp.float32),
    ]
  scratch += [pltpu.VMEM_SHARED((NP, dh), jnp.float32)]
  scratch += [pltpu.SemaphoreType.DMA] * nbuf
  ept = E // NS if feature_split else E1P // (NC * NS)
  nchunk = ept // chunk
  assert ept % chunk == 0
  if nbuf == 3:
    m = (nchunk - 2) // 3
    assert nchunk == 3 * m + 2
  else:
    m = (nchunk - 1) // 2
    assert nchunk == 2 * m + 1

  def body(table, src, dst, zacc, out, *rest):
    bufs = [tuple(rest[3 * b:3 * b + 3]) for b in range(nbuf)]
    acc = rest[3 * nbuf]
    sems = rest[3 * nbuf + 1:]
    c = lax.axis_index("c")
    s = lax.axis_index("s")
    rbase = s * ROWS_PT

    pltpu.sync_copy(zacc.at[pl.ds(rbase, ROWS_PT)], acc.at[pl.ds(rbase, ROWS_PT)])
    plsc.subcore_barrier()

    if feature_split:
      ebase = s * ept
      roff = c * N
    else:
      ebase = (c * NS + s) * ept

    def start(i, b):
      sb, db, rb = bufs[b]
      off = ebase + i * chunk
      pltpu.sync_copy(src.at[pl.ds(off, chunk)], sb)
      pltpu.sync_copy(dst.at[pl.ds(off, chunk)], db)
      if feature_split:
        for j in range(chunk // 16):
          sl = pl.ds(j * 16, 16)
          sb[sl] = sb[sl] + roff
      pltpu.async_copy(table.at[sb], rb, sems[b])

    def drain(b):
      sb, db, rb = bufs[b]
      pltpu.make_async_copy(table.at[sb], rb, sems[b]).wait()
      pltpu.sync_copy(rb, acc.at[db], add=True)

    if nbuf == 3:
      start(0, 0)
      start(1, 1)

      def loop_body(k, carry):
        start(3 * k + 2, 2)
        drain(0)
        start(3 * k + 3, 0)
        drain(1)
        start(3 * k + 4, 1)
        drain(2)
        return carry

      lax.fori_loop(0, m, loop_body, 0)
      drain(0)
      drain(1)
    else:
      start(0, 0)

      def loop_body(k, carry):
        start(2 * k + 1, 1)
        drain(0)
        start(2 * k + 2, 0)
        drain(1)
        return carry

      lax.fori_loop(0, m, loop_body, 0)
      drain(0)
    plsc.subcore_barrier()

    pltpu.sync_copy(acc.at[pl.ds(rbase, ROWS_PT)],
                    out.at[pl.ds(c * NP + rbase, ROWS_PT)])

  return functools.partial(
      pl.kernel, mesh=mesh, out_type=out_type, scratch_types=scratch
  )(body)


CCHUNK = 80


@functools.lru_cache(maxsize=None)
def _make_sc_cnt():
  mesh = plsc.VectorSubcoreMesh(core_axis_name="c", subcore_axis_name="s",
                                num_cores=NC, num_subcores=NS)
  out_type = jax.ShapeDtypeStruct((NC * NP, 128), jnp.float32)
  scratch = [
      pltpu.VMEM((CCHUNK,), jnp.int32),
      pltpu.VMEM((CCHUNK,), jnp.int32),
      pltpu.VMEM((CCHUNK, 128), jnp.float32),
      pltpu.VMEM_SHARED((NP, 128), jnp.float32),
      pltpu.SemaphoreType.DMA,
      pltpu.SemaphoreType.DMA,
  ]
  ept = E1P // (NC * NS)
  nchunk = ept // CCHUNK

  pairs = (nchunk - 1) // 2
  assert nchunk == 2 * pairs + 1

  def body(dst, zacc, ones_h, out, db0, db1, ones_v, acc, sem0, sem1):
    c = lax.axis_index("c")
    s = lax.axis_index("s")
    rbase = s * ROWS_PT
    pltpu.sync_copy(zacc.at[pl.ds(rbase, ROWS_PT)], acc.at[pl.ds(rbase, ROWS_PT)])
    pltpu.sync_copy(ones_h, ones_v)
    plsc.subcore_barrier()
    ebase = (c * NS + s) * ept

    def start(i, db, sem):
      pltpu.sync_copy(dst.at[pl.ds(ebase + i * CCHUNK, CCHUNK)], db)
      pltpu.async_copy(ones_v, acc.at[db], sem, add=True)

    def drain(db, sem):
      pltpu.make_async_copy(ones_v, acc.at[db], sem).wait()

    start(0, db0, sem0)

    def pair_body(k, carry):
      start(2 * k + 1, db1, sem1)
      drain(db0, sem0)
      start(2 * k + 2, db0, sem0)
      drain(db1, sem1)
      return carry

    lax.fori_loop(0, pairs, pair_body, 0)
    drain(db0, sem0)
    plsc.subcore_barrier()
    pltpu.sync_copy(acc.at[pl.ds(rbase, ROWS_PT)],
                    out.at[pl.ds(c * NP + rbase, ROWS_PT)])

  return functools.partial(
      pl.kernel, mesh=mesh, out_type=out_type, scratch_types=scratch
  )(body)


def _tca_body(x_ref, wl_ref, wr_ref, wres_ref, resb_ref,
              ycat_ref, xr_ref, res_ref):
  xb = x_ref[...]
  y = jnp.dot(xb, wl_ref[...], preferred_element_type=jnp.float32)
  ycat_ref[0] = y[:, : D_HID // 2]
  ycat_ref[1] = y[:, D_HID // 2:]
  xr_ref[...] = jnp.dot(xb, wr_ref[...], preferred_element_type=jnp.float32)
  res_ref[...] = (jnp.dot(xb, wres_ref[...], preferred_element_type=jnp.float32)
                  + resb_ref[...])


_tca = pl.pallas_call(
    _tca_body,
    grid=(NB,),
    in_specs=[
        pl.BlockSpec((RBLK, D_IN), lambda i: (i, 0)),
        pl.BlockSpec((D_IN, D_HID), lambda i: (0, 0)),
        pl.BlockSpec((D_IN, D_HID), lambda i: (0, 0)),
        pl.BlockSpec((D_IN, D_OUT), lambda i: (0, 0)),
        pl.BlockSpec((1, D_OUT), lambda i: (0, 0)),
    ],
    out_specs=[
        pl.BlockSpec((2, RBLK, D_HID // 2), lambda i: (0, i, 0)),
        pl.BlockSpec((RBLK, D_HID), lambda i: (i, 0)),
        pl.BlockSpec((RBLK, D_OUT), lambda i: (i, 0)),
    ],
    out_shape=[
        jax.ShapeDtypeStruct((2, N, D_HID // 2), jnp.float32),
        jax.ShapeDtypeStruct((N, D_HID), jnp.float32),
        jax.ShapeDtypeStruct((N, D_OUT), jnp.float32),
    ],
)


def _tcb_body(sum_ref, cnt_ref, xr_ref, bl0_ref, g_ref, b_ref,
              wl1_ref, wr1_ref, ycat_ref, hr_ref, h_s, stat_s):
  p = pl.program_id(0)
  i = pl.program_id(1)

  @pl.when(p == 0)
  def _():
    @pl.when(i == 0)
    def _():
      stat_s[...] = jnp.zeros_like(stat_s)

    inv = 1.0 / jnp.maximum(cnt_ref[0, :, 0:1] + cnt_ref[1, :, 0:1], 1.0)
    t = (jnp.concatenate([sum_ref[0], sum_ref[1]], axis=1) * inv
         + bl0_ref[...] + xr_ref[...])
    nrm = jnp.maximum(jnp.sqrt(jnp.sum(t * t, axis=1, keepdims=True)), 1e-12)
    t = t / nrm
    h_s[pl.ds(i * RBLK, RBLK), :] = t
    stat_s[0:1, :] += jnp.sum(t, axis=0, keepdims=True)
    stat_s[1:2, :] += jnp.sum(t * t, axis=0, keepdims=True)

  @pl.when(p == 1)
  def _():
    mu = stat_s[0:1, :] / N
    var = stat_s[1:2, :] / N - mu * mu
    t = h_s[pl.ds(i * RBLK, RBLK), :]
    t = g_ref[...] * (t - mu) * lax.rsqrt(var + 1e-5) + b_ref[...]
    t = jnp.maximum(t, 0.0)
    ycat_ref[...] = jnp.dot(t, wl1_ref[...], preferred_element_type=jnp.float32)
    hr_ref[...] = jnp.dot(t, wr1_ref[...], preferred_element_type=jnp.float32)


_tcb = pl.pallas_call(
    _tcb_body,
    grid=(2, NB),
    in_specs=[
        pl.BlockSpec((2, RBLK, D_HID // 2), lambda p, i: (0, i, 0)),
        pl.BlockSpec((2, RBLK, 8), lambda p, i: (0, i, 0)),
        pl.BlockSpec((RBLK, D_HID), lambda p, i: (i, 0)),
        pl.BlockSpec((1, D_HID), lambda p, i: (0, 0)),
        pl.BlockSpec((1, D_HID), lambda p, i: (0, 0)),
        pl.BlockSpec((1, D_HID), lambda p, i: (0, 0)),
        pl.BlockSpec((D_HID, D_OUT), lambda p, i: (0, 0)),
        pl.BlockSpec((D_HID, D_OUT), lambda p, i: (0, 0)),
    ],
    out_specs=[
        pl.BlockSpec((RBLK, D_OUT), lambda p, i: (i, 0)),
        pl.BlockSpec((RBLK, D_OUT), lambda p, i: (i, 0)),
    ],
    out_shape=[
        jax.ShapeDtypeStruct((N, D_OUT), jnp.float32),
        jax.ShapeDtypeStruct((N, D_OUT), jnp.float32),
    ],
    scratch_shapes=[
        pltpu.VMEM((N, D_HID), jnp.float32),
        pltpu.VMEM((2, D_HID), jnp.float32),
    ],
)


def _tcc_body(sum_ref, cnt_ref, hr_ref, res_ref, bl1_ref, out_ref):
  inv = 1.0 / jnp.maximum(cnt_ref[0, :, 0:1] + cnt_ref[1, :, 0:1], 1.0)
  t = (sum_ref[0] + sum_ref[1]) * inv + bl1_ref[...] + hr_ref[...]
  nrm = jnp.maximum(jnp.sqrt(jnp.sum(t * t, axis=1, keepdims=True)), 1e-12)
  t = t / nrm + res_ref[...]
  nrm = jnp.maximum(jnp.sqrt(jnp.sum(t * t, axis=1, keepdims=True)), 1e-12)
  out_ref[...] = t / nrm


_tcc = pl.pallas_call(
    _tcc_body,
    grid=(NB,),
    in_specs=[
        pl.BlockSpec((2, RBLK, D_OUT), lambda i: (0, i, 0)),
        pl.BlockSpec((2, RBLK, 8), lambda i: (0, i, 0)),
        pl.BlockSpec((RBLK, D_OUT), lambda i: (i, 0)),
        pl.BlockSpec((RBLK, D_OUT), lambda i: (i, 0)),
        pl.BlockSpec((1, D_OUT), lambda i: (0, 0)),
    ],
    out_specs=pl.BlockSpec((RBLK, D_OUT), lambda i: (i, 0)),
    out_shape=jax.ShapeDtypeStruct((N, D_OUT), jnp.float32),
)


def kernel(x, edge_index, Wl0, bl0, Wr0, gamma0, beta0, Wl1, bl1, Wr1,
           resW, resb):
  src = edge_index[0]
  dst = edge_index[1]
  src1 = jnp.concatenate([src, jnp.zeros((E1P - E,), jnp.int32)])
  dst1 = jnp.concatenate([dst, jnp.full((E1P - E,), N, jnp.int32)])

  ycat0, xr0, res = _tca(x, Wl0.T, Wr0.T, resW.T, resb.reshape(1, -1))

  zacc = jnp.zeros((NP, 128), jnp.float32)
  ones_h = jnp.ones((CCHUNK, 128), jnp.float32)
  cntcat = _make_sc_cnt()(dst1, zacc, ones_h)
  cnt8 = cntcat.reshape(NC, NP, 128)[:, :N, :8]

  sum0 = _make_sc_agg(128, CHUNK, True)(
      ycat0.reshape(NC * N, 128), src, dst, zacc)
  sum0 = sum0.reshape(NC, NP, 128)[:, :N, :]

  y1, hr1 = _tcb(sum0, cnt8, xr0,
                 bl0.reshape(1, -1), gamma0.reshape(1, -1),
                 beta0.reshape(1, -1), Wl1.T, Wr1.T)

  (sum1) = _make_sc_agg(128, CCHUNK, False)(y1, src1, dst1, zacc)
  sum1 = sum1.reshape(NC, NP, 128)[:, :N, :]

  return _tcc(sum1, cnt8, hr1, res, bl1.reshape(1, -1))
